# Initial kernel scaffold; baseline (speedup 1.0000x reference)
import jax
import jax.numpy as jnp
from jax.experimental import pallas as pl


def kernel(x, edge_index, edge_rel, edge_neg, node_batch, rel_emb):
    raise NotImplementedError("write your pallas kernel here")


# speedup vs baseline: 4.7285x; 4.7285x over previous
import functools

import jax
import jax.numpy as jnp
from jax import lax
from jax.experimental import pallas as pl
from jax.experimental.pallas import tpu as pltpu
from jax.experimental.pallas import tpu_sc as plsc

GAMMA = 12.0
NUM_GRAPHS = 1024
NC, NS, L = 2, 16, 16
NW = NC * NS


def _make_sc_edge_kernel(N, E, D, R, G):
    EW = E // NW
    C = 80
    n_chunks = EW // C
    assert EW % C == 0 and D % L == 0 and G % L == 0

    mesh = plsc.VectorSubcoreMesh(core_axis_name="c", subcore_axis_name="s")

    @functools.partial(
        pl.kernel,
        mesh=mesh,
        out_type=jax.ShapeDtypeStruct((NW, G), jnp.float32),
        compiler_params=pltpu.CompilerParams(needs_layout_passes=False),
        scratch_types=[
            pltpu.VMEM((C,), jnp.int32),
            pltpu.VMEM((C,), jnp.int32),
            pltpu.VMEM((C,), jnp.int32),
            pltpu.VMEM((C,), jnp.int32),
            pltpu.VMEM((C, D), jnp.float32),
            pltpu.VMEM((C, D), jnp.float32),
            pltpu.VMEM((C, D), jnp.float32),
            pltpu.VMEM((N,), jnp.int32),
            pltpu.VMEM((G,), jnp.float32),
            pltpu.SemaphoreType.DMA,
        ],
    )
    def sc_kernel(x_hbm, hid_hbm, tid_hbm, rid_hbm, neg_hbm, nb_hbm, rel_hbm,
                  out_hbm,
                  hid_v, tid_v, rid_v, neg_v, hrow_v, trow_v, rrow_v,
                  nb_v, min_v, sem):
        wid = lax.axis_index("s") * NC + lax.axis_index("c")

        pltpu.sync_copy(nb_hbm, nb_v)

        inf_vec = jnp.full((L,), jnp.inf, dtype=jnp.float32)
        mask0 = lax.iota(jnp.int32, L) == 0

        def init_body(i, _):
            min_v[pl.ds(i * L, L)] = inf_vec
            return _

        lax.fori_loop(0, G // L, init_body, None)

        def group_body(g, _):
            hid16 = hid_v[pl.ds(g * L, L)]
            neg16 = neg_v[pl.ds(g * L, L)]
            seg16 = plsc.load_gather(nb_v, [hid16])
            for l in range(L):
                e = g * L + l
                acc = jnp.zeros((L,), jnp.float32)
                for k in range(D // L):
                    h = hrow_v[e, pl.ds(k * L, L)]
                    r = rrow_v[e, pl.ds(k * L, L)]
                    t = trow_v[e, pl.ds(k * L, L)]
                    acc = acc + jnp.abs(h + r - t)
                dist = jnp.sum(acc)
                sval = lax.select(neg16[l] == 1, dist - GAMMA, GAMMA - dist)
                idx = jnp.full((L,), seg16[l], jnp.int32)
                cur = plsc.load_gather(min_v, [idx])[0]
                new = lax.select(sval < cur, sval, cur)
                plsc.store_scatter(min_v, [idx],
                                   jnp.full((L,), new, jnp.float32),
                                   mask=mask0)
            return _

        def chunk_body(c, _):
            base = wid * EW + c * C
            pltpu.sync_copy(hid_hbm.at[pl.ds(base, C)], hid_v)
            pltpu.sync_copy(tid_hbm.at[pl.ds(base, C)], tid_v)
            pltpu.sync_copy(rid_hbm.at[pl.ds(base, C)], rid_v)
            pltpu.sync_copy(neg_hbm.at[pl.ds(base, C)], neg_v)
            pltpu.async_copy(x_hbm.at[hid_v], hrow_v, sem).wait()
            pltpu.async_copy(x_hbm.at[tid_v], trow_v, sem).wait()
            pltpu.async_copy(rel_hbm.at[rid_v], rrow_v, sem).wait()
            lax.fori_loop(0, C // L, group_body, None)
            return _

        lax.fori_loop(0, n_chunks, chunk_body, None)

        pltpu.sync_copy(min_v, out_hbm.at[wid])

    return sc_kernel


def _make_tc_combine(G):
    def body(p_ref, o_ref):
        m = p_ref[0]
        for i in range(1, NW):
            m = jnp.minimum(m, p_ref[i])
        o_ref[...] = jnp.where(jnp.isinf(m), m, jax.nn.sigmoid(m))

    return pl.pallas_call(
        body,
        out_shape=jax.ShapeDtypeStruct((8, G // 8), jnp.float32),
    )


def kernel(x, edge_index, edge_rel, edge_neg, node_batch, rel_emb):
    N, D = x.shape
    E = edge_index.shape[1]
    R = rel_emb.shape[0]
    G = NUM_GRAPHS

    sc_kernel = _make_sc_edge_kernel(N, E, D, R, G)
    partial_mins = sc_kernel(
        x,
        edge_index[0],
        edge_index[1],
        edge_rel,
        edge_neg,
        node_batch,
        rel_emb,
    )
    combine = _make_tc_combine(G)
    out = combine(partial_mins.reshape(NW, 8, G // 8))
    return out.reshape(G)


# speedup vs baseline: 10.2596x; 2.1697x over previous
import functools

import jax
import jax.numpy as jnp
from jax import lax
from jax.experimental import pallas as pl
from jax.experimental.pallas import tpu as pltpu
from jax.experimental.pallas import tpu_sc as plsc

GAMMA = 12.0
NUM_GRAPHS = 1024
NC, NS, L = 2, 16, 16
NW = NC * NS


def _make_sc_edge_kernel(N, E, D, R, G):
    EW = E // NW
    C = 80
    n_chunks = EW // C
    assert EW % C == 0 and C % L == 0 and D % L == 0 and G % L == 0
    assert n_chunks % 2 == 1

    mesh = plsc.VectorSubcoreMesh(core_axis_name="c", subcore_axis_name="s")

    @functools.partial(
        pl.kernel,
        mesh=mesh,
        out_type=jax.ShapeDtypeStruct((NW, G), jnp.float32),
        compiler_params=pltpu.CompilerParams(needs_layout_passes=False),
        scratch_types=[
            pltpu.VMEM((4, C), jnp.int32),
            pltpu.VMEM((4, C), jnp.int32),
            pltpu.VMEM((C, D), jnp.float32),
            pltpu.VMEM((C, D), jnp.float32),
            pltpu.VMEM((C, D), jnp.float32),
            pltpu.VMEM((C, D), jnp.float32),
            pltpu.VMEM((C, D), jnp.float32),
            pltpu.VMEM((C, D), jnp.float32),
            pltpu.VMEM((N,), jnp.int32),
            pltpu.VMEM((G,), jnp.float32),
            pltpu.SemaphoreType.DMA,
            pltpu.SemaphoreType.DMA,
            pltpu.SemaphoreType.DMA,
            pltpu.SemaphoreType.DMA,
        ],
    )
    def sc_kernel(x_hbm, edata_hbm, nb_hbm, rel_hbm, out_hbm,
                  eb0, eb1, h0, t0, r0, h1, t1, r1, nb_v, min_v,
                  isem0, isem1, rsem0, rsem1):
        wid = lax.axis_index("s") * NC + lax.axis_index("c")
        cbase = wid * n_chunks

        ebufs = (eb0, eb1)
        rows = ((h0, t0, r0), (h1, t1, r1))
        isems = (isem0, isem1)
        rsems = (rsem0, rsem1)

        pltpu.sync_copy(nb_hbm, nb_v)

        inf_vec = jnp.full((L,), jnp.inf, dtype=jnp.float32)
        mask0 = lax.iota(jnp.int32, L) == 0

        def init_body(i, _):
            min_v[pl.ds(i * L, L)] = inf_vec
            return _

        lax.fori_loop(0, G // L, init_body, None)

        def idx_copy(c, b):
            return pltpu.make_async_copy(
                edata_hbm.at[cbase + c], ebufs[b], isems[b])

        def row_copies(b):
            eb = ebufs[b]
            hb, tb, rb = rows[b]
            return (
                pltpu.make_async_copy(x_hbm.at[eb.at[0]], hb, rsems[b]),
                pltpu.make_async_copy(x_hbm.at[eb.at[1]], tb, rsems[b]),
                pltpu.make_async_copy(rel_hbm.at[eb.at[2]], rb, rsems[b]),
            )

        def compute_chunk(b):
            eb = ebufs[b]
            hb, tb, rb = rows[b]

            def group_body(g, _):
                hid16 = eb[0, pl.ds(g * L, L)]
                neg16 = eb[3, pl.ds(g * L, L)]
                seg16 = plsc.load_gather(nb_v, [hid16])
                for l in range(L):
                    e = g * L + l
                    acc = jnp.zeros((L,), jnp.float32)
                    for k in range(D // L):
                        h = hb[e, pl.ds(k * L, L)]
                        r = rb[e, pl.ds(k * L, L)]
                        t = tb[e, pl.ds(k * L, L)]
                        acc = acc + jnp.abs(h + r - t)
                    dist = jnp.sum(acc)
                    sval = lax.select(neg16[l] == 1,
                                      dist - GAMMA, GAMMA - dist)
                    idx = jnp.full((L,), seg16[l], jnp.int32)
                    cur = plsc.load_gather(min_v, [idx])[0]
                    new = lax.select(sval < cur, sval, cur)
                    plsc.store_scatter(min_v, [idx],
                                       jnp.full((L,), new, jnp.float32),
                                       mask=mask0)
                return _

            lax.fori_loop(0, C // L, group_body, None)

        def chunk_step(c, b, steady):
            for cp in row_copies(b):
                cp.wait()
            if steady:
                @pl.when(c + 2 < n_chunks)
                def _():
                    idx_copy(c + 2, b).start()
                idx_copy(c + 1, 1 - b).wait()
                for cp in row_copies(1 - b):
                    cp.start()
            compute_chunk(b)

        idx_copy(0, 0).start()
        idx_copy(1, 1).start()
        idx_copy(0, 0).wait()
        for cp in row_copies(0):
            cp.start()

        def pair_body(c2, _):
            c = 2 * c2
            chunk_step(c, 0, True)
            chunk_step(c + 1, 1, True)
            return _

        lax.fori_loop(0, (n_chunks - 1) // 2, pair_body, None)
        chunk_step(n_chunks - 1, 0, False)

        pltpu.sync_copy(min_v, out_hbm.at[wid])

    return sc_kernel


def _make_tc_combine(G):
    def body(p_ref, o_ref):
        m = p_ref[0]
        for i in range(1, NW):
            m = jnp.minimum(m, p_ref[i])
        o_ref[...] = jnp.where(jnp.isinf(m), m, jax.nn.sigmoid(m))

    return pl.pallas_call(
        body,
        out_shape=jax.ShapeDtypeStruct((8, G // 8), jnp.float32),
    )


def kernel(x, edge_index, edge_rel, edge_neg, node_batch, rel_emb):
    N, D = x.shape
    E = edge_index.shape[1]
    G = NUM_GRAPHS
    C = 80
    n_chunks = E // NW // C

    edata = jnp.stack(
        [edge_index[0].reshape(-1, C), edge_index[1].reshape(-1, C),
         edge_rel.reshape(-1, C), edge_neg.reshape(-1, C)],
        axis=1,
    )

    sc_kernel = _make_sc_edge_kernel(N, E, D, rel_emb.shape[0], G)
    partial_mins = sc_kernel(x, edata, node_batch, rel_emb)
    combine = _make_tc_combine(G)
    out = combine(partial_mins.reshape(NW, 8, G // 8))
    return out.reshape(G)


# speedup vs baseline: 10.9014x; 1.0626x over previous
import functools

import jax
import jax.numpy as jnp
from jax import lax
from jax.experimental import pallas as pl
from jax.experimental.pallas import tpu as pltpu
from jax.experimental.pallas import tpu_sc as plsc

GAMMA = 12.0
NUM_GRAPHS = 1024
NC, NS, L = 2, 16, 16
NW = NC * NS


def _make_sc_edge_kernel(N, E, D, R, G):
    EW = E // NW
    C = 80
    n_chunks = EW // C
    Dp = D // 2
    assert EW % C == 0 and C % L == 0 and Dp % L == 0 and G % L == 0
    assert n_chunks % 2 == 1

    mesh = plsc.VectorSubcoreMesh(core_axis_name="c", subcore_axis_name="s")

    @functools.partial(
        pl.kernel,
        mesh=mesh,
        out_type=jax.ShapeDtypeStruct((NW, G), jnp.float32),
        compiler_params=pltpu.CompilerParams(needs_layout_passes=False,
                                             use_tc_tiling_on_sc=False),
        scratch_types=[
            pltpu.VMEM((4, C), jnp.int32),
            pltpu.VMEM((4, C), jnp.int32),
            pltpu.VMEM((C, Dp), jnp.float32),
            pltpu.VMEM((C, Dp), jnp.float32),
            pltpu.VMEM((C, Dp), jnp.float32),
            pltpu.VMEM((C, Dp), jnp.float32),
            pltpu.VMEM((C, Dp), jnp.float32),
            pltpu.VMEM((C, Dp), jnp.float32),
            pltpu.VMEM((N,), jnp.int32),
            pltpu.VMEM((G,), jnp.float32),
            pltpu.SemaphoreType.DMA,
            pltpu.SemaphoreType.DMA,
            pltpu.SemaphoreType.DMA,
            pltpu.SemaphoreType.DMA,
        ],
    )
    def sc_kernel(x_hbm, edata_hbm, nb_hbm, rel_hbm, out_hbm,
                  eb0, eb1, h0, t0, r0, h1, t1, r1, nb_v, min_v,
                  isem0, isem1, rsem0, rsem1):
        wid = lax.axis_index("s") * NC + lax.axis_index("c")
        cbase = wid * n_chunks

        ebufs = (eb0, eb1)
        rows = ((h0, t0, r0), (h1, t1, r1))
        isems = (isem0, isem1)
        rsems = (rsem0, rsem1)

        pltpu.sync_copy(nb_hbm, nb_v)

        inf_vec = jnp.full((L,), jnp.inf, dtype=jnp.float32)
        mask0 = lax.iota(jnp.int32, L) == 0

        def init_body(i, _):
            min_v[pl.ds(i * L, L)] = inf_vec
            return _

        lax.fori_loop(0, G // L, init_body, None)

        def idx_copy(c, b):
            return pltpu.make_async_copy(
                edata_hbm.at[cbase + c], ebufs[b], isems[b])

        def row_copies(b):
            eb = ebufs[b]
            hb, tb, rb = rows[b]
            return (
                pltpu.make_async_copy(x_hbm.at[eb.at[0]], hb, rsems[b]),
                pltpu.make_async_copy(x_hbm.at[eb.at[1]], tb, rsems[b]),
                pltpu.make_async_copy(rel_hbm.at[eb.at[2]], rb, rsems[b]),
            )

        def compute_chunk(b):
            eb = ebufs[b]
            hb, tb, rb = rows[b]

            def group_body(g, _):
                hid16 = eb[0, pl.ds(g * L, L)]
                neg16 = eb[3, pl.ds(g * L, L)]
                seg16 = plsc.load_gather(nb_v, [hid16])
                for l in range(L):
                    e = g * L + l
                    acc = jnp.zeros((L,), jnp.float32)
                    for k in range(Dp // L):
                        h = plsc.bitcast(hb[e, pl.ds(k * L, L)], jnp.bfloat16)
                        r = plsc.bitcast(rb[e, pl.ds(k * L, L)], jnp.bfloat16)
                        t = plsc.bitcast(tb[e, pl.ds(k * L, L)], jnp.bfloat16)
                        lo, hi = plsc.unpack(jnp.abs(h + r - t),
                                             format=plsc.PackFormat.INTERLEAVED)
                        acc = acc + (lo + hi)
                    dist = jnp.sum(acc)
                    sval = lax.select(neg16[l] == 1,
                                      dist - GAMMA, GAMMA - dist)
                    idx = jnp.full((L,), seg16[l], jnp.int32)
                    cur = plsc.load_gather(min_v, [idx])[0]
                    new = lax.select(sval < cur, sval, cur)
                    plsc.store_scatter(min_v, [idx],
                                       jnp.full((L,), new, jnp.float32),
                                       mask=mask0)
                return _

            lax.fori_loop(0, C // L, group_body, None)

        def chunk_step(c, b, steady):
            for cp in row_copies(b):
                cp.wait()
            if steady:
                @pl.when(c + 2 < n_chunks)
                def _():
                    idx_copy(c + 2, b).start()
                idx_copy(c + 1, 1 - b).wait()
                for cp in row_copies(1 - b):
                    cp.start()
            compute_chunk(b)

        idx_copy(0, 0).start()
        idx_copy(1, 1).start()
        idx_copy(0, 0).wait()
        for cp in row_copies(0):
            cp.start()

        def pair_body(c2, _):
            c = 2 * c2
            chunk_step(c, 0, True)
            chunk_step(c + 1, 1, True)
            return _

        lax.fori_loop(0, (n_chunks - 1) // 2, pair_body, None)
        chunk_step(n_chunks - 1, 0, False)

        pltpu.sync_copy(min_v, out_hbm.at[wid])

    return sc_kernel


def _make_tc_combine(G):
    def body(p_ref, o_ref):
        m = p_ref[0]
        for i in range(1, NW):
            m = jnp.minimum(m, p_ref[i])
        o_ref[...] = jnp.where(jnp.isinf(m), m, jax.nn.sigmoid(m))

    return pl.pallas_call(
        body,
        out_shape=jax.ShapeDtypeStruct((8, G // 8), jnp.float32),
    )


def kernel(x, edge_index, edge_rel, edge_neg, node_batch, rel_emb):
    N, D = x.shape
    E = edge_index.shape[1]
    G = NUM_GRAPHS
    C = 80
    n_chunks = E // NW // C

    edata = jnp.stack(
        [edge_index[0].reshape(-1, C), edge_index[1].reshape(-1, C),
         edge_rel.reshape(-1, C), edge_neg.reshape(-1, C)],
        axis=1,
    )

    xp = lax.bitcast_convert_type(
        x.astype(jnp.bfloat16).reshape(N, D // 2, 2), jnp.float32)
    relp = lax.bitcast_convert_type(
        rel_emb.astype(jnp.bfloat16).reshape(rel_emb.shape[0], D // 2, 2),
        jnp.float32)

    sc_kernel = _make_sc_edge_kernel(N, E, D, rel_emb.shape[0], G)
    partial_mins = sc_kernel(xp, edata, node_batch, relp)
    combine = _make_tc_combine(G)
    out = combine(partial_mins.reshape(NW, 8, G // 8))
    return out.reshape(G)


# speedup vs baseline: 14.5873x; 1.3381x over previous
import functools

import jax
import jax.numpy as jnp
from jax import lax
from jax.experimental import pallas as pl
from jax.experimental.pallas import tpu as pltpu
from jax.experimental.pallas import tpu_sc as plsc

GAMMA = 12.0
NUM_GRAPHS = 1024
NC, NS, L = 2, 16, 16
NW = NC * NS


def _make_sc_edge_kernel(N, E, D, R, G):
    EW = E // NW
    C = 80
    n_chunks = EW // C
    Dp = D // 2
    assert EW % C == 0 and C % L == 0 and Dp % L == 0 and G % L == 0
    assert n_chunks % 2 == 1

    mesh = plsc.VectorSubcoreMesh(core_axis_name="c", subcore_axis_name="s")

    @functools.partial(
        pl.kernel,
        mesh=mesh,
        out_type=jax.ShapeDtypeStruct((NW, G), jnp.float32),
        compiler_params=pltpu.CompilerParams(needs_layout_passes=False,
                                             use_tc_tiling_on_sc=False),
        scratch_types=[
            pltpu.VMEM((4, C), jnp.int32),
            pltpu.VMEM((4, C), jnp.int32),
            pltpu.VMEM((C, Dp), jnp.float32),
            pltpu.VMEM((C, Dp), jnp.float32),
            pltpu.VMEM((C, Dp), jnp.float32),
            pltpu.VMEM((C, Dp), jnp.float32),
            pltpu.VMEM((C, Dp), jnp.float32),
            pltpu.VMEM((C, Dp), jnp.float32),
            pltpu.VMEM((N,), jnp.int32),
            pltpu.VMEM((G,), jnp.float32),
            pltpu.VMEM((L,), jnp.float32),
            pltpu.SemaphoreType.DMA,
            pltpu.SemaphoreType.DMA,
            pltpu.SemaphoreType.DMA,
            pltpu.SemaphoreType.DMA,
        ],
    )
    def sc_kernel(x_hbm, edata_hbm, nb_hbm, rel_hbm, out_hbm,
                  eb0, eb1, h0, t0, r0, h1, t1, r1, nb_v, min_v, dist_v,
                  isem0, isem1, rsem0, rsem1):
        wid = lax.axis_index("s") * NC + lax.axis_index("c")
        cbase = wid * n_chunks

        ebufs = (eb0, eb1)
        rows = ((h0, t0, r0), (h1, t1, r1))
        isems = (isem0, isem1)
        rsems = (rsem0, rsem1)

        pltpu.sync_copy(nb_hbm, nb_v)

        inf_vec = jnp.full((L,), jnp.inf, dtype=jnp.float32)
        mask0 = lax.iota(jnp.int32, L) == 0

        def init_body(i, _):
            min_v[pl.ds(i * L, L)] = inf_vec
            return _

        lax.fori_loop(0, G // L, init_body, None)

        def idx_copy(c, b):
            return pltpu.make_async_copy(
                edata_hbm.at[cbase + c], ebufs[b], isems[b])

        def row_copies(b):
            eb = ebufs[b]
            hb, tb, rb = rows[b]
            return (
                pltpu.make_async_copy(x_hbm.at[eb.at[0]], hb, rsems[b]),
                pltpu.make_async_copy(x_hbm.at[eb.at[1]], tb, rsems[b]),
                pltpu.make_async_copy(rel_hbm.at[eb.at[2]], rb, rsems[b]),
            )

        def compute_chunk(b):
            eb = ebufs[b]
            hb, tb, rb = rows[b]
            mask15 = lax.iota(jnp.int32, L) == (L - 1)

            def group_body(g, _):
                hid16 = eb[0, pl.ds(g * L, L)]
                neg16 = eb[3, pl.ds(g * L, L)]
                seg16 = plsc.load_gather(nb_v, [hid16])
                for l in range(L):
                    e = g * L + l
                    acc = jnp.zeros((2 * L,), jnp.bfloat16)
                    for k in range(Dp // L):
                        h = plsc.bitcast(hb[e, pl.ds(k * L, L)], jnp.bfloat16)
                        r = plsc.bitcast(rb[e, pl.ds(k * L, L)], jnp.bfloat16)
                        t = plsc.bitcast(tb[e, pl.ds(k * L, L)], jnp.bfloat16)
                        acc = acc + jnp.abs(h + r - t)
                    lo, hi = plsc.unpack(acc,
                                         format=plsc.PackFormat.INTERLEAVED)
                    cum = jnp.cumsum(lo + hi)
                    plsc.store_scatter(dist_v,
                                       [jnp.full((L,), l, jnp.int32)],
                                       cum, mask=mask15)
                dist16 = dist_v[...]
                sval16 = jnp.where(neg16 == 1,
                                   dist16 - GAMMA, GAMMA - dist16)
                cur = plsc.load_gather(min_v, [seg16])
                new = jnp.minimum(sval16, cur)

                def retry_cond(pend):
                    return jnp.any(pend)

                def retry_body(pend):
                    plsc.store_scatter(min_v, [seg16], new, mask=pend)
                    chk = plsc.load_gather(min_v, [seg16])
                    return chk > new

                lax.while_loop(retry_cond, retry_body,
                               jnp.full((L,), True))
                return _

            lax.fori_loop(0, C // L, group_body, None)

        def chunk_step(c, b, steady):
            for cp in row_copies(b):
                cp.wait()
            if steady:
                @pl.when(c + 2 < n_chunks)
                def _():
                    idx_copy(c + 2, b).start()
                idx_copy(c + 1, 1 - b).wait()
                for cp in row_copies(1 - b):
                    cp.start()
            compute_chunk(b)

        idx_copy(0, 0).start()
        idx_copy(1, 1).start()
        idx_copy(0, 0).wait()
        for cp in row_copies(0):
            cp.start()

        def pair_body(c2, _):
            c = 2 * c2
            chunk_step(c, 0, True)
            chunk_step(c + 1, 1, True)
            return _

        lax.fori_loop(0, (n_chunks - 1) // 2, pair_body, None)
        chunk_step(n_chunks - 1, 0, False)

        pltpu.sync_copy(min_v, out_hbm.at[wid])

    return sc_kernel


def _make_tc_combine(G):
    def body(p_ref, o_ref):
        m = p_ref[0]
        for i in range(1, NW):
            m = jnp.minimum(m, p_ref[i])
        o_ref[...] = jnp.where(jnp.isinf(m), m, jax.nn.sigmoid(m))

    return pl.pallas_call(
        body,
        out_shape=jax.ShapeDtypeStruct((8, G // 8), jnp.float32),
    )


def kernel(x, edge_index, edge_rel, edge_neg, node_batch, rel_emb):
    N, D = x.shape
    E = edge_index.shape[1]
    G = NUM_GRAPHS
    C = 80
    n_chunks = E // NW // C

    edata = jnp.stack(
        [edge_index[0].reshape(-1, C), edge_index[1].reshape(-1, C),
         edge_rel.reshape(-1, C), edge_neg.reshape(-1, C)],
        axis=1,
    )

    xp = lax.bitcast_convert_type(
        x.astype(jnp.bfloat16).reshape(N, D // 2, 2), jnp.float32)
    relp = lax.bitcast_convert_type(
        rel_emb.astype(jnp.bfloat16).reshape(rel_emb.shape[0], D // 2, 2),
        jnp.float32)

    sc_kernel = _make_sc_edge_kernel(N, E, D, rel_emb.shape[0], G)
    partial_mins = sc_kernel(xp, edata, node_batch, relp)
    combine = _make_tc_combine(G)
    out = combine(partial_mins.reshape(NW, 8, G // 8))
    return out.reshape(G)


# speedup vs baseline: 17.2246x; 1.1808x over previous
import functools

import jax
import jax.numpy as jnp
from jax import lax
from jax.experimental import pallas as pl
from jax.experimental.pallas import tpu as pltpu
from jax.experimental.pallas import tpu_sc as plsc

GAMMA = 12.0
NUM_GRAPHS = 1024
NC, NS, L = 2, 16, 16
NW = NC * NS


def _make_sc_edge_kernel(N, E, D, R, G):
    EW = E // NW
    C = 80
    n_chunks = EW // C
    Dp = D // 2
    assert EW % C == 0 and C % L == 0 and Dp % L == 0 and G % L == 0
    assert n_chunks % 2 == 1

    mesh = plsc.VectorSubcoreMesh(core_axis_name="c", subcore_axis_name="s")

    @functools.partial(
        pl.kernel,
        mesh=mesh,
        out_type=jax.ShapeDtypeStruct((NW, G), jnp.float32),
        compiler_params=pltpu.CompilerParams(needs_layout_passes=False,
                                             use_tc_tiling_on_sc=False),
        scratch_types=[
            pltpu.VMEM((4, C), jnp.int32),
            pltpu.VMEM((4, C), jnp.int32),
            pltpu.VMEM((C, Dp), jnp.float32),
            pltpu.VMEM((C, Dp), jnp.float32),
            pltpu.VMEM((C, Dp), jnp.float32),
            pltpu.VMEM((C, Dp), jnp.float32),
            pltpu.VMEM((C, Dp), jnp.float32),
            pltpu.VMEM((C, Dp), jnp.float32),
            pltpu.VMEM((N,), jnp.int32),
            pltpu.VMEM((G,), jnp.float32),
            pltpu.VMEM((L * 17,), jnp.float32),
            pltpu.SemaphoreType.DMA,
            pltpu.SemaphoreType.DMA,
            pltpu.SemaphoreType.DMA,
            pltpu.SemaphoreType.DMA,
        ],
    )
    def sc_kernel(x_hbm, edata_hbm, nb_hbm, rel_hbm, out_hbm,
                  eb0, eb1, h0, t0, r0, h1, t1, r1, nb_v, min_v, mat_v,
                  isem0, isem1, rsem0, rsem1):
        wid = lax.axis_index("s") * NC + lax.axis_index("c")
        cbase = wid * n_chunks

        ebufs = (eb0, eb1)
        rows = ((h0, t0, r0), (h1, t1, r1))
        isems = (isem0, isem1)
        rsems = (rsem0, rsem1)

        pltpu.sync_copy(nb_hbm, nb_v)

        inf_vec = jnp.full((L,), jnp.inf, dtype=jnp.float32)
        mask0 = lax.iota(jnp.int32, L) == 0

        def init_body(i, _):
            min_v[pl.ds(i * L, L)] = inf_vec
            return _

        lax.fori_loop(0, G // L, init_body, None)

        def idx_copy(c, b):
            return pltpu.make_async_copy(
                edata_hbm.at[cbase + c], ebufs[b], isems[b])

        def row_copies(b):
            eb = ebufs[b]
            hb, tb, rb = rows[b]
            return (
                pltpu.make_async_copy(x_hbm.at[eb.at[0]], hb, rsems[b]),
                pltpu.make_async_copy(x_hbm.at[eb.at[1]], tb, rsems[b]),
                pltpu.make_async_copy(rel_hbm.at[eb.at[2]], rb, rsems[b]),
            )

        def compute_chunk(b):
            eb = ebufs[b]
            hb, tb, rb = rows[b]
            iota = lax.iota(jnp.int32, L)
            col_idx = iota * 17

            def group_body(g, _):
                hid16 = eb[0, pl.ds(g * L, L)]
                neg16 = eb[3, pl.ds(g * L, L)]
                seg16 = plsc.load_gather(nb_v, [hid16])
                for l in range(L):
                    e = g * L + l
                    acc = jnp.zeros((2 * L,), jnp.bfloat16)
                    for k in range(Dp // L):
                        h = plsc.bitcast(hb[e, pl.ds(k * L, L)], jnp.bfloat16)
                        r = plsc.bitcast(rb[e, pl.ds(k * L, L)], jnp.bfloat16)
                        t = plsc.bitcast(tb[e, pl.ds(k * L, L)], jnp.bfloat16)
                        acc = acc + jnp.abs(h + r - t)
                    lo, hi = plsc.unpack(acc,
                                         format=plsc.PackFormat.INTERLEAVED)
                    plsc.store_scatter(mat_v, [iota + (17 * l)], lo + hi)
                cols = [plsc.load_gather(mat_v, [col_idx + j])
                        for j in range(L)]
                while len(cols) > 1:
                    cols = [a + b for a, b in zip(cols[0::2], cols[1::2])]
                dist16 = cols[0]
                sval16 = jnp.where(neg16 == 1,
                                   dist16 - GAMMA, GAMMA - dist16)
                cur = plsc.load_gather(min_v, [seg16])
                new = jnp.minimum(sval16, cur)

                def retry_cond(pend):
                    return jnp.any(pend)

                def retry_body(pend):
                    plsc.store_scatter(min_v, [seg16], new, mask=pend)
                    chk = plsc.load_gather(min_v, [seg16])
                    return chk > new

                lax.while_loop(retry_cond, retry_body,
                               jnp.full((L,), True))
                return _

            lax.fori_loop(0, C // L, group_body, None)

        def chunk_step(c, b, steady):
            for cp in row_copies(b):
                cp.wait()
            if steady:
                @pl.when(c + 2 < n_chunks)
                def _():
                    idx_copy(c + 2, b).start()
                idx_copy(c + 1, 1 - b).wait()
                for cp in row_copies(1 - b):
                    cp.start()
            compute_chunk(b)

        idx_copy(0, 0).start()
        idx_copy(1, 1).start()
        idx_copy(0, 0).wait()
        for cp in row_copies(0):
            cp.start()

        def pair_body(c2, _):
            c = 2 * c2
            chunk_step(c, 0, True)
            chunk_step(c + 1, 1, True)
            return _

        lax.fori_loop(0, (n_chunks - 1) // 2, pair_body, None)
        chunk_step(n_chunks - 1, 0, False)

        pltpu.sync_copy(min_v, out_hbm.at[wid])

    return sc_kernel


def _make_tc_combine(G):
    def body(p_ref, o_ref):
        m = p_ref[0]
        for i in range(1, NW):
            m = jnp.minimum(m, p_ref[i])
        o_ref[...] = jnp.where(jnp.isinf(m), m, jax.nn.sigmoid(m))

    return pl.pallas_call(
        body,
        out_shape=jax.ShapeDtypeStruct((8, G // 8), jnp.float32),
    )


def kernel(x, edge_index, edge_rel, edge_neg, node_batch, rel_emb):
    N, D = x.shape
    E = edge_index.shape[1]
    G = NUM_GRAPHS
    C = 80
    n_chunks = E // NW // C

    edata = jnp.stack(
        [edge_index[0].reshape(-1, C), edge_index[1].reshape(-1, C),
         edge_rel.reshape(-1, C), edge_neg.reshape(-1, C)],
        axis=1,
    )

    xp = lax.bitcast_convert_type(
        x.astype(jnp.bfloat16).reshape(N, D // 2, 2), jnp.float32)
    relp = lax.bitcast_convert_type(
        rel_emb.astype(jnp.bfloat16).reshape(rel_emb.shape[0], D // 2, 2),
        jnp.float32)

    sc_kernel = _make_sc_edge_kernel(N, E, D, rel_emb.shape[0], G)
    partial_mins = sc_kernel(xp, edata, node_batch, relp)
    combine = _make_tc_combine(G)
    out = combine(partial_mins.reshape(NW, 8, G // 8))
    return out.reshape(G)
